# baseline (speedup 1.0000x reference)
import functools

import jax
import jax.numpy as jnp
from jax import lax
from jax.experimental import pallas as pl
from jax.experimental.pallas import tpu as pltpu
from jax.experimental.pallas import tpu_sc as plsc

N_NODES = 10000
D = 128
E_PER_TYPE = 45714
N_TYPES = 7

NC = 2
NS = 16

BLK = 128
NBLK_FULL = 23
NBLK_HALF = 12
N_ITEMS = 8
E_PAD_FULL = NS * NBLK_FULL * BLK
E_PAD_HALF6 = 2 * NS * NBLK_HALF * BLK
NROWS = 10240
RPT = NROWS // NS
N_TRASH = NROWS - N_NODES
DEPTH = 2


def _sc_segment_sums(x, src_f, dst_f, src_h, dst_h, zrows, zer640, one128):
    mesh = plsc.VectorSubcoreMesh(core_axis_name="c", subcore_axis_name="s")

    @functools.partial(
        pl.kernel,
        out_type=[
            jax.ShapeDtypeStruct((N_ITEMS, NROWS, D), jnp.float32),
            jax.ShapeDtypeStruct((N_ITEMS * NROWS,), jnp.float32),
        ],
        mesh=mesh,
        scratch_types=[
            pltpu.VMEM_SHARED((NROWS, D), jnp.float32),
            pltpu.VMEM_SHARED((NROWS,), jnp.float32),
            pltpu.VMEM((NBLK_FULL, BLK), jnp.int32),
            pltpu.VMEM((NBLK_FULL, BLK), jnp.int32),
            pltpu.VMEM((NBLK_HALF, BLK), jnp.int32),
            pltpu.VMEM((NBLK_HALF, BLK), jnp.int32),
            pltpu.VMEM((DEPTH, BLK, D), jnp.float32),
            pltpu.VMEM((BLK,), jnp.float32),
            pltpu.SemaphoreType.DMA,
            pltpu.SemaphoreType.DMA,
            pltpu.SemaphoreType.DMA,
            pltpu.SemaphoreType.DMA,
            pltpu.SemaphoreType.DMA,
            pltpu.SemaphoreType.DMA,
            pltpu.SemaphoreType.DMA,
            pltpu.SemaphoreType.DMA,
            pltpu.SemaphoreType.DMA,
            pltpu.SemaphoreType.DMA,
        ],
    )
    def k(x_hbm, srcf_hbm, dstf_hbm, srch_hbm, dsth_hbm,
          zrows_hbm, z640_hbm, o128_hbm,
          acc_out, deg_out,
          acc_s, deg_s, srcf_v, dstf_v, srch_v, dsth_v, rows_v, o128_v,
          sem_g0, sem_g1, sem_g2, sem_g3,
          sem_s0, sem_s1, sem_s2, sem_s3, sem_d, sem_z):
        cid = lax.axis_index("c")
        sid = lax.axis_index("s")
        base = sid * RPT

        pltpu.sync_copy(o128_hbm, o128_v)

        sem_g = [sem_g0, sem_g1, sem_g2, sem_g3]
        sem_s = [sem_s0, sem_s1, sem_s2, sem_s3]

        def process_item(it, src_hbm, dst_hbm, ai, nblk, src_v, dst_v):
            @pl.when(cid == (it % NC))
            def _():
                zds = [pltpu.async_copy(zrows_hbm,
                                        acc_s.at[pl.ds(base, RPT)], sem_z),
                       pltpu.async_copy(z640_hbm, deg_s.at[pl.ds(base, RPT)],
                                        sem_z)]
                pltpu.sync_copy(src_hbm.at[ai, sid], src_v)
                pltpu.sync_copy(dst_hbm.at[ai, sid], dst_v)
                for d in zds:
                    d.wait()
                plsc.subcore_barrier()

                gd = [None] * DEPTH
                sd = [None] * DEPTH
                dd = []
                for a in range(min(DEPTH - 1, nblk)):
                    gd[a] = pltpu.async_copy(x_hbm.at[src_v.at[a]],
                                             rows_v.at[a], sem_g[a])
                for j in range(nblk):
                    cur = j % DEPTH
                    ahead = j + DEPTH - 1
                    if ahead < nblk:
                        ab = ahead % DEPTH
                        if sd[ab] is not None:
                            sd[ab].wait()
                        gd[ab] = pltpu.async_copy(x_hbm.at[src_v.at[ahead]],
                                                  rows_v.at[ab], sem_g[ab])
                    gd[cur].wait()
                    sd[cur] = pltpu.async_copy(rows_v.at[cur],
                                               acc_s.at[dst_v.at[j]],
                                               sem_s[cur], add=True)
                    dd.append(pltpu.async_copy(o128_v, deg_s.at[dst_v.at[j]],
                                               sem_d, add=True))
                for d in sd:
                    if d is not None:
                        d.wait()
                for d in dd:
                    d.wait()
                plsc.subcore_barrier()

                wo = [pltpu.async_copy(acc_s.at[pl.ds(base, RPT)],
                                       acc_out.at[it, pl.ds(base, RPT)], sem_z),
                      pltpu.async_copy(deg_s.at[pl.ds(base, RPT)],
                                       deg_out.at[pl.ds(it * NROWS + base, RPT)],
                                       sem_z)]
                for d in wo:
                    d.wait()

        for it in range(6):
            process_item(it, srcf_hbm, dstf_hbm, it, NBLK_FULL, srcf_v, dstf_v)
        for hh in range(2):
            process_item(6 + hh, srch_hbm, dsth_hbm, hh, NBLK_HALF,
                         srch_v, dsth_v)

    return k(x, src_f, dst_f, src_h, dst_h, zrows, zer640, one128)


def _tc_combine(acc, deg3, x, w_neigh, w_self, b):
    R = 400
    grid = (N_NODES // R,)

    def body(acc_ref, deg_ref, x_ref, wn_ref, ws_ref, b_ref, out_ref):
        degv = deg_ref[...]
        accv = acc_ref[...]
        ws = jnp.sum(ws_ref[...], axis=0)
        res = jnp.dot(x_ref[...], ws, preferred_element_type=jnp.float32)
        res = res + jnp.sum(b_ref[...], axis=0)[None, :]
        cols = []
        for t in range(N_TYPES):
            if t < 6:
                a, dg = accv[t], degv[t]
            else:
                a, dg = accv[6] + accv[7], degv[6] + degv[7]
            cols.append(a / jnp.maximum(dg, 1.0))
        aggn = jnp.concatenate(cols, axis=1)
        wn = wn_ref[...].reshape(N_TYPES * D, D)
        res = res + jnp.dot(aggn, wn, preferred_element_type=jnp.float32)
        out_ref[...] = res * (1.0 / N_TYPES)

    return pl.pallas_call(
        body,
        grid=grid,
        in_specs=[
            pl.BlockSpec((N_ITEMS, R, D), lambda i: (0, i, 0)),
            pl.BlockSpec((N_ITEMS, R, 1), lambda i: (0, i, 0)),
            pl.BlockSpec((R, D), lambda i: (i, 0)),
            pl.BlockSpec((N_TYPES, D, D), lambda i: (0, 0, 0)),
            pl.BlockSpec((N_TYPES, D, D), lambda i: (0, 0, 0)),
            pl.BlockSpec((N_TYPES, D), lambda i: (0, 0)),
        ],
        out_specs=pl.BlockSpec((R, D), lambda i: (i, 0)),
        out_shape=jax.ShapeDtypeStruct((N_NODES, D), jnp.float32),
    )(acc, deg3, x, w_neigh, w_self, b)


def _spread_pads(src_t, dst_t, e_pad):
    pad = e_pad - E_PER_TYPE
    ar = jnp.arange(pad, dtype=src_t.dtype)
    pad_src = (ar * 37) % N_NODES
    pad_dst = N_NODES + (ar % N_TRASH)
    return (jnp.concatenate([src_t, pad_src]),
            jnp.concatenate([dst_t, pad_dst]))


def kernel(x, edge_index_c2c, edge_index_c2d, edge_index_c2e, edge_index_codoc,
           edge_index_comention, edge_index_d2e, edge_index_ent,
           W_neigh, W_self, b):
    edge_lists = [edge_index_c2c, edge_index_c2d, edge_index_c2e,
                  edge_index_codoc, edge_index_comention, edge_index_d2e,
                  edge_index_ent]
    src = jnp.stack([ei[0] for ei in edge_lists])
    dst = jnp.stack([ei[1] for ei in edge_lists])

    padf = E_PAD_FULL - E_PER_TYPE
    arf = jnp.arange(padf, dtype=src.dtype)
    pad_srcf = jnp.broadcast_to((arf * 37) % N_NODES, (6, padf))
    pad_dstf = jnp.broadcast_to(N_NODES + (arf % N_TRASH), (6, padf))
    src_f = jnp.concatenate([src[:6], pad_srcf], axis=1)
    dst_f = jnp.concatenate([dst[:6], pad_dstf], axis=1)

    src_6, dst_6 = _spread_pads(src[6], dst[6], E_PAD_HALF6)

    src_fr = src_f.reshape(6, NS, NBLK_FULL, BLK)
    dst_fr = dst_f.reshape(6, NS, NBLK_FULL, BLK)
    src_hr = src_6.reshape(2, NS, NBLK_HALF, BLK)
    dst_hr = dst_6.reshape(2, NS, NBLK_HALF, BLK)

    zrows = jnp.zeros((RPT, D), jnp.float32)
    zer640 = jnp.zeros((RPT,), jnp.float32)
    one128 = jnp.ones((BLK,), jnp.float32)

    acc, deg = _sc_segment_sums(x, src_fr, dst_fr, src_hr, dst_hr,
                                zrows, zer640, one128)
    deg3 = deg.reshape(N_ITEMS, NROWS, 1)
    return _tc_combine(acc, deg3, x, W_neigh, W_self, b)


# speedup vs baseline: 1.0394x; 1.0394x over previous
import functools

import jax
import jax.numpy as jnp
from jax import lax
from jax.experimental import pallas as pl
from jax.experimental.pallas import tpu as pltpu
from jax.experimental.pallas import tpu_sc as plsc

N_NODES = 10000
D = 128
E_PER_TYPE = 45714
N_TYPES = 7

NC = 2
NS = 16

BLK = 128
NBLK_FULL = 23
NBLK_HALF = 12
N_ITEMS = 8
E_PAD_FULL = NS * NBLK_FULL * BLK
E_PAD_HALF6 = 2 * NS * NBLK_HALF * BLK
NROWS = 10240
RPT = NROWS // NS
N_TRASH = NROWS - N_NODES


def _sc_segment_sums(x, src_f, dst_f, src_h, dst_h, zrows, zer640, one128):
    mesh = plsc.VectorSubcoreMesh(core_axis_name="c", subcore_axis_name="s")

    @functools.partial(
        pl.kernel,
        out_type=[
            jax.ShapeDtypeStruct((N_ITEMS, NROWS, D), jnp.float32),
            jax.ShapeDtypeStruct((N_ITEMS * NROWS,), jnp.float32),
        ],
        mesh=mesh,
        scratch_types=[
            pltpu.VMEM_SHARED((NROWS, D), jnp.float32),
            pltpu.VMEM_SHARED((NROWS,), jnp.float32),
            pltpu.VMEM((NBLK_FULL, BLK), jnp.int32),
            pltpu.VMEM((NBLK_FULL, BLK), jnp.int32),
            pltpu.VMEM((NBLK_HALF, BLK), jnp.int32),
            pltpu.VMEM((NBLK_HALF, BLK), jnp.int32),
            pltpu.VMEM((2, BLK, D), jnp.float32),
            pltpu.VMEM((BLK,), jnp.float32),
            pltpu.SemaphoreType.DMA,
            pltpu.SemaphoreType.DMA,
            pltpu.SemaphoreType.DMA,
            pltpu.SemaphoreType.DMA,
            pltpu.SemaphoreType.DMA,
            pltpu.SemaphoreType.DMA,
        ],
    )
    def k(x_hbm, srcf_hbm, dstf_hbm, srch_hbm, dsth_hbm,
          zrows_hbm, z640_hbm, o128_hbm,
          acc_out, deg_out,
          acc_s, deg_s, srcf_v, dstf_v, srch_v, dsth_v, rows_v, o128_v,
          sem_g0, sem_g1, sem_s0, sem_s1, sem_d, sem_z):
        cid = lax.axis_index("c")
        sid = lax.axis_index("s")
        base = sid * RPT

        pltpu.sync_copy(o128_hbm, o128_v)

        sem_g = [sem_g0, sem_g1]
        sem_s = [sem_s0, sem_s1]

        def process_item(it, src_hbm, dst_hbm, ai, nblk, src_v, dst_v):
            @pl.when(cid == (it % NC))
            def _():
                zds = [pltpu.async_copy(zrows_hbm.at[sid],
                                        acc_s.at[pl.ds(base, RPT)], sem_z),
                       pltpu.async_copy(z640_hbm, deg_s.at[pl.ds(base, RPT)],
                                        sem_z)]
                pltpu.sync_copy(src_hbm.at[ai, sid], src_v)
                pltpu.sync_copy(dst_hbm.at[ai, sid], dst_v)
                for d in zds:
                    d.wait()
                plsc.subcore_barrier()

                gd = [None, None]
                sd = [None, None]
                dd = []
                gd[0] = pltpu.async_copy(x_hbm.at[src_v.at[0]],
                                         rows_v.at[0], sem_g[0])
                for j in range(nblk):
                    cur, nxt = j % 2, (j + 1) % 2
                    if j + 1 < nblk:
                        if sd[nxt] is not None:
                            sd[nxt].wait()
                        gd[nxt] = pltpu.async_copy(x_hbm.at[src_v.at[j + 1]],
                                                   rows_v.at[nxt], sem_g[nxt])
                    gd[cur].wait()
                    sd[cur] = pltpu.async_copy(rows_v.at[cur],
                                               acc_s.at[dst_v.at[j]],
                                               sem_s[cur], add=True)
                    dd.append(pltpu.async_copy(o128_v, deg_s.at[dst_v.at[j]],
                                               sem_d, add=True))
                for d in sd:
                    if d is not None:
                        d.wait()
                for d in dd:
                    d.wait()
                plsc.subcore_barrier()

                wo = [pltpu.async_copy(acc_s.at[pl.ds(base, RPT)],
                                       acc_out.at[it, pl.ds(base, RPT)], sem_z),
                      pltpu.async_copy(deg_s.at[pl.ds(base, RPT)],
                                       deg_out.at[pl.ds(it * NROWS + base, RPT)],
                                       sem_z)]
                for d in wo:
                    d.wait()

        for it in range(6):
            process_item(it, srcf_hbm, dstf_hbm, it, NBLK_FULL, srcf_v, dstf_v)
        for hh in range(2):
            process_item(6 + hh, srch_hbm, dsth_hbm, hh, NBLK_HALF,
                         srch_v, dsth_v)

    return k(x, src_f, dst_f, src_h, dst_h, zrows, zer640, one128)


def _tc_combine(acc, deg3, x, w_neigh, w_self, b):
    R = 1000
    grid = (N_NODES // R,)

    def body(acc_ref, deg_ref, x_ref, wn_ref, ws_ref, b_ref, out_ref):
        degv = deg_ref[...]
        accv = acc_ref[...]
        ws = jnp.sum(ws_ref[...], axis=0)
        res = jnp.dot(x_ref[...], ws, preferred_element_type=jnp.float32)
        res = res + jnp.sum(b_ref[...], axis=0)[None, :]
        for t in range(N_TYPES):
            if t < 6:
                a, dg = accv[t], degv[t]
            else:
                a, dg = accv[6] + accv[7], degv[6] + degv[7]
            aggn = a / jnp.maximum(dg, 1.0)
            res = res + jnp.dot(aggn, wn_ref[t],
                                preferred_element_type=jnp.float32)
        out_ref[...] = res * (1.0 / N_TYPES)

    return pl.pallas_call(
        body,
        grid=grid,
        in_specs=[
            pl.BlockSpec((N_ITEMS, R, D), lambda i: (0, i, 0)),
            pl.BlockSpec((N_ITEMS, R, 1), lambda i: (0, i, 0)),
            pl.BlockSpec((R, D), lambda i: (i, 0)),
            pl.BlockSpec((N_TYPES, D, D), lambda i: (0, 0, 0)),
            pl.BlockSpec((N_TYPES, D, D), lambda i: (0, 0, 0)),
            pl.BlockSpec((N_TYPES, D), lambda i: (0, 0)),
        ],
        out_specs=pl.BlockSpec((R, D), lambda i: (i, 0)),
        out_shape=jax.ShapeDtypeStruct((N_NODES, D), jnp.float32),
    )(acc, deg3, x, w_neigh, w_self, b)


def _spread_pads(src_t, dst_t, e_pad):
    pad = e_pad - E_PER_TYPE
    ar = jnp.arange(pad, dtype=src_t.dtype)
    pad_src = (ar * 37) % N_NODES
    pad_dst = N_NODES + (ar % N_TRASH)
    return (jnp.concatenate([src_t, pad_src]),
            jnp.concatenate([dst_t, pad_dst]))


def kernel(x, edge_index_c2c, edge_index_c2d, edge_index_c2e, edge_index_codoc,
           edge_index_comention, edge_index_d2e, edge_index_ent,
           W_neigh, W_self, b):
    edge_lists = [edge_index_c2c, edge_index_c2d, edge_index_c2e,
                  edge_index_codoc, edge_index_comention, edge_index_d2e,
                  edge_index_ent]
    src = jnp.stack([ei[0] for ei in edge_lists])
    dst = jnp.stack([ei[1] for ei in edge_lists])

    padf = E_PAD_FULL - E_PER_TYPE
    arf = jnp.arange(padf, dtype=src.dtype)
    pad_srcf = jnp.broadcast_to((arf * 37) % N_NODES, (6, padf))
    pad_dstf = jnp.broadcast_to(N_NODES + (arf % N_TRASH), (6, padf))
    src_f = jnp.concatenate([src[:6], pad_srcf], axis=1)
    dst_f = jnp.concatenate([dst[:6], pad_dstf], axis=1)

    src_6, dst_6 = _spread_pads(src[6], dst[6], E_PAD_HALF6)

    src_fr = src_f.reshape(6, NS, NBLK_FULL, BLK)
    dst_fr = dst_f.reshape(6, NS, NBLK_FULL, BLK)
    src_hr = src_6.reshape(2, NS, NBLK_HALF, BLK)
    dst_hr = dst_6.reshape(2, NS, NBLK_HALF, BLK)

    zrows = jnp.zeros((NS, RPT, D), jnp.float32)
    zer640 = jnp.zeros((RPT,), jnp.float32)
    one128 = jnp.ones((BLK,), jnp.float32)

    acc, deg = _sc_segment_sums(x, src_fr, dst_fr, src_hr, dst_hr,
                                zrows, zer640, one128)
    deg3 = deg.reshape(N_ITEMS, NROWS, 1)
    return _tc_combine(acc, deg3, x, W_neigh, W_self, b)


# speedup vs baseline: 1.2527x; 1.2053x over previous
import functools

import jax
import jax.numpy as jnp
from jax import lax
from jax.experimental import pallas as pl
from jax.experimental.pallas import tpu as pltpu
from jax.experimental.pallas import tpu_sc as plsc

N_NODES = 10000
D = 128
E_PER_TYPE = 45714
N_TYPES = 7

NC = 2
NS = 16

BLK = 128
NBLK_FULL = 23
NBLK_HALF = 12
N_ITEMS = 8
E_PAD_FULL = NS * NBLK_FULL * BLK
E_PAD_HALF6 = 2 * NS * NBLK_HALF * BLK
NROWS = 10240
RPT = NROWS // NS
N_TRASH = NROWS - N_NODES


def _sc_segment_sums(x, src_f, dst_f, src_h, dst_h, zrows, zer640, one128):
    mesh = plsc.VectorSubcoreMesh(core_axis_name="c", subcore_axis_name="s")

    @functools.partial(
        pl.kernel,
        out_type=[
            jax.ShapeDtypeStruct((N_ITEMS, NROWS, D), jnp.float32),
            jax.ShapeDtypeStruct((N_ITEMS * NROWS,), jnp.float32),
        ],
        mesh=mesh,
        scratch_types=[
            pltpu.VMEM_SHARED((NROWS, D), jnp.float32),
            pltpu.VMEM_SHARED((NROWS,), jnp.float32),
            pltpu.VMEM((NBLK_FULL, BLK), jnp.int32),
            pltpu.VMEM((NBLK_FULL, BLK), jnp.int32),
            pltpu.VMEM((NBLK_HALF, BLK), jnp.int32),
            pltpu.VMEM((NBLK_HALF, BLK), jnp.int32),
            pltpu.VMEM((2, BLK, D), jnp.float32),
            pltpu.VMEM((BLK,), jnp.float32),
            pltpu.SemaphoreType.DMA,
            pltpu.SemaphoreType.DMA,
            pltpu.SemaphoreType.DMA,
            pltpu.SemaphoreType.DMA,
            pltpu.SemaphoreType.DMA,
            pltpu.SemaphoreType.DMA,
        ],
    )
    def k(x_hbm, srcf_hbm, dstf_hbm, srch_hbm, dsth_hbm,
          zrows_hbm, z640_hbm, o128_hbm,
          acc_out, deg_out,
          acc_s, deg_s, srcf_v, dstf_v, srch_v, dsth_v, rows_v, o128_v,
          sem_g0, sem_g1, sem_s0, sem_s1, sem_d, sem_z):
        cid = lax.axis_index("c")
        sid = lax.axis_index("s")
        base = sid * RPT

        pltpu.sync_copy(o128_hbm, o128_v)

        sem_g = [sem_g0, sem_g1]
        sem_s = [sem_s0, sem_s1]

        def process_item(it, src_hbm, dst_hbm, ai, nblk, src_v, dst_v):
            @pl.when(cid == (it % NC))
            def _():
                zds = [pltpu.async_copy(zrows_hbm.at[sid],
                                        acc_s.at[pl.ds(base, RPT)], sem_z),
                       pltpu.async_copy(z640_hbm, deg_s.at[pl.ds(base, RPT)],
                                        sem_z)]
                pltpu.sync_copy(src_hbm.at[ai, sid], src_v)
                pltpu.sync_copy(dst_hbm.at[ai, sid], dst_v)
                for d in zds:
                    d.wait()
                plsc.subcore_barrier()

                gd = [None, None]
                sd = [None, None]
                dd = []
                gd[0] = pltpu.async_copy(x_hbm.at[src_v.at[0]],
                                         rows_v.at[0], sem_g[0])
                for j in range(nblk):
                    cur, nxt = j % 2, (j + 1) % 2
                    if j + 1 < nblk:
                        if sd[nxt] is not None:
                            sd[nxt].wait()
                        gd[nxt] = pltpu.async_copy(x_hbm.at[src_v.at[j + 1]],
                                                   rows_v.at[nxt], sem_g[nxt])
                    gd[cur].wait()
                    sd[cur] = pltpu.async_copy(rows_v.at[cur],
                                               acc_s.at[dst_v.at[j]],
                                               sem_s[cur], add=True)
                    dd.append(pltpu.async_copy(o128_v, deg_s.at[dst_v.at[j]],
                                               sem_d, add=True))
                for d in sd:
                    if d is not None:
                        d.wait()
                for d in dd:
                    d.wait()
                plsc.subcore_barrier()

                wo = [pltpu.async_copy(acc_s.at[pl.ds(base, RPT)],
                                       acc_out.at[it, pl.ds(base, RPT)], sem_z),
                      pltpu.async_copy(deg_s.at[pl.ds(base, RPT)],
                                       deg_out.at[pl.ds(it * NROWS + base, RPT)],
                                       sem_z)]
                for d in wo:
                    d.wait()

        for it in range(6):
            process_item(it, srcf_hbm, dstf_hbm, it, NBLK_FULL, srcf_v, dstf_v)
        for hh in range(2):
            process_item(6 + hh, srch_hbm, dsth_hbm, hh, NBLK_HALF,
                         srch_v, dsth_v)

    return k(x, src_f, dst_f, src_h, dst_h, zrows, zer640, one128)


def _tc_combine(acc, deg_t, x, w_neigh, w_self, b):
    R = 1000
    grid = (N_NODES // R,)

    def body(acc_ref, deg_ref, x_ref, wn_ref, ws_ref, b_ref, out_ref):
        degv = deg_ref[...]
        accv = acc_ref[...]
        ws = jnp.sum(ws_ref[...], axis=0)
        res = jnp.dot(x_ref[...], ws, preferred_element_type=jnp.float32)
        res = res + jnp.sum(b_ref[...], axis=0)[None, :]
        for t in range(N_TYPES):
            if t < 6:
                a, dg = accv[t], degv[:, t:t + 1]
            else:
                a = accv[6] + accv[7]
                dg = degv[:, 6:7] + degv[:, 7:8]
            aggn = a / jnp.maximum(dg, 1.0)
            res = res + jnp.dot(aggn, wn_ref[t],
                                preferred_element_type=jnp.float32)
        out_ref[...] = res * (1.0 / N_TYPES)

    return pl.pallas_call(
        body,
        grid=grid,
        in_specs=[
            pl.BlockSpec((N_ITEMS, R, D), lambda i: (0, i, 0)),
            pl.BlockSpec((R, N_ITEMS), lambda i: (i, 0)),
            pl.BlockSpec((R, D), lambda i: (i, 0)),
            pl.BlockSpec((N_TYPES, D, D), lambda i: (0, 0, 0)),
            pl.BlockSpec((N_TYPES, D, D), lambda i: (0, 0, 0)),
            pl.BlockSpec((N_TYPES, D), lambda i: (0, 0)),
        ],
        out_specs=pl.BlockSpec((R, D), lambda i: (i, 0)),
        out_shape=jax.ShapeDtypeStruct((N_NODES, D), jnp.float32),
    )(acc, deg_t, x, w_neigh, w_self, b)


def _spread_pads(src_t, dst_t, e_pad):
    pad = e_pad - E_PER_TYPE
    ar = jnp.arange(pad, dtype=src_t.dtype)
    pad_src = (ar * 37) % N_NODES
    pad_dst = N_NODES + (ar % N_TRASH)
    return (jnp.concatenate([src_t, pad_src]),
            jnp.concatenate([dst_t, pad_dst]))


def kernel(x, edge_index_c2c, edge_index_c2d, edge_index_c2e, edge_index_codoc,
           edge_index_comention, edge_index_d2e, edge_index_ent,
           W_neigh, W_self, b):
    edge_lists = [edge_index_c2c, edge_index_c2d, edge_index_c2e,
                  edge_index_codoc, edge_index_comention, edge_index_d2e,
                  edge_index_ent]
    src = jnp.stack([ei[0] for ei in edge_lists])
    dst = jnp.stack([ei[1] for ei in edge_lists])

    padf = E_PAD_FULL - E_PER_TYPE
    arf = jnp.arange(padf, dtype=src.dtype)
    pad_srcf = jnp.broadcast_to((arf * 37) % N_NODES, (6, padf))
    pad_dstf = jnp.broadcast_to(N_NODES + (arf % N_TRASH), (6, padf))
    src_f = jnp.concatenate([src[:6], pad_srcf], axis=1)
    dst_f = jnp.concatenate([dst[:6], pad_dstf], axis=1)

    src_6, dst_6 = _spread_pads(src[6], dst[6], E_PAD_HALF6)

    src_fr = src_f.reshape(6, NS, NBLK_FULL, BLK)
    dst_fr = dst_f.reshape(6, NS, NBLK_FULL, BLK)
    src_hr = src_6.reshape(2, NS, NBLK_HALF, BLK)
    dst_hr = dst_6.reshape(2, NS, NBLK_HALF, BLK)

    zrows = jnp.zeros((NS, RPT, D), jnp.float32)
    zer640 = jnp.zeros((RPT,), jnp.float32)
    one128 = jnp.ones((BLK,), jnp.float32)

    acc, deg = _sc_segment_sums(x, src_fr, dst_fr, src_hr, dst_hr,
                                zrows, zer640, one128)
    deg_t = deg.reshape(N_ITEMS, NROWS).T
    return _tc_combine(acc, deg_t, x, W_neigh, W_self, b)
